# baseline (speedup 1.0000x reference)
import functools

import jax
import jax.numpy as jnp
from jax import lax
from jax.experimental import pallas as pl
from jax.experimental.pallas import tpu as pltpu
from jax.experimental.pallas import tpu_sc as plsc

DIM = 32
L = 16
NW = 32
CHUNK = 128


def _build(B):
    b_per_w = B // NW
    n_chunks = b_per_w // CHUNK
    n_groups = b_per_w // L
    mesh = plsc.VectorSubcoreMesh(core_axis_name="c", subcore_axis_name="s")

    @functools.partial(
        pl.kernel,
        out_type=jax.ShapeDtypeStruct((B,), jnp.float32),
        mesh=mesh,
        scratch_types=[
            pltpu.VMEM((b_per_w,), jnp.int32),
            pltpu.VMEM((b_per_w,), jnp.int32),
            pltpu.VMEM((b_per_w,), jnp.int32),
            pltpu.VMEM((b_per_w, DIM), jnp.float32),
            pltpu.VMEM((b_per_w, DIM), jnp.float32),
            pltpu.VMEM((b_per_w, DIM), jnp.float32),
            pltpu.VMEM((3 * DIM,), jnp.float32),
            pltpu.VMEM((b_per_w,), jnp.float32),
            pltpu.SemaphoreType.DMA,
        ],
        compiler_params=pltpu.CompilerParams(
            needs_layout_passes=False, use_tc_tiling_on_sc=False),
    )
    def sc_kernel(a0_hbm, a1_hbm, src_hbm, table_hbm, rel_hbm, out_hbm,
                  i0_v, i1_v, i2_v, r0_v, r1_v, r2_v, rv_v, out_v, sem):
        wid = lax.axis_index("s") * 2 + lax.axis_index("c")
        base = wid * b_per_w

        pltpu.sync_copy(a0_hbm.at[pl.ds(base, b_per_w)], i0_v)
        pltpu.sync_copy(a1_hbm.at[pl.ds(base, b_per_w)], i1_v)
        pltpu.sync_copy(src_hbm.at[pl.ds(base, b_per_w)], i2_v)
        pltpu.sync_copy(rel_hbm, rv_v)

        rd_lo = (rv_v[pl.ds(0, L)] * rv_v[pl.ds(DIM, L)]
                 * rv_v[pl.ds(2 * DIM, L)])
        rd_hi = (rv_v[pl.ds(L, L)] * rv_v[pl.ds(DIM + L, L)]
                 * rv_v[pl.ds(2 * DIM + L, L)])

        copies = []
        for (iv, rv) in ((i0_v, r0_v), (i1_v, r1_v), (i2_v, r2_v)):
            for j in range(n_chunks):
                s = j * CHUNK
                copies.append(pltpu.async_copy(
                    table_hbm.at[iv.at[pl.ds(s, CHUNK)]],
                    rv.at[pl.ds(s, CHUNK)], sem))
        for c in copies:
            c.wait()

        lane = lax.iota(jnp.int32, L)

        def group(g, carry):
            base_e = g * L
            num = jnp.zeros((L,), jnp.float32)
            saa = jnp.zeros((L,), jnp.float32)
            see = jnp.zeros((L,), jnp.float32)
            for j in range(L):
                e = base_e + j
                a_lo = r0_v[e, pl.ds(0, L)]
                a_hi = r0_v[e, pl.ds(L, L)]
                b_lo = r1_v[e, pl.ds(0, L)]
                b_hi = r1_v[e, pl.ds(L, L)]
                t_lo = r2_v[e, pl.ds(0, L)]
                t_hi = r2_v[e, pl.ds(L, L)]
                u_lo = a_lo * b_lo
                u_hi = a_hi * b_hi
                w_lo = t_lo * rd_lo
                w_hi = t_hi * rd_hi
                num_s = jnp.sum(u_lo * w_lo + u_hi * w_hi)
                saa_s = jnp.sum(w_lo * w_lo + w_hi * w_hi)
                see_s = jnp.sum(u_lo * u_lo + u_hi * u_hi)
                m = lane == j
                num = jnp.where(m, num_s, num)
                saa = jnp.where(m, saa_s, saa)
                see = jnp.where(m, see_s, see)
            x = jnp.maximum(saa * see, jnp.float32(1e-24))
            i = plsc.bitcast(x, jnp.int32)
            y = plsc.bitcast(jnp.int32(0x5F3759DF) - (i >> 1), jnp.float32)
            for _ in range(3):
                y = y * (jnp.float32(1.5) - jnp.float32(0.5) * x * y * y)
            out_v[pl.ds(base_e, L)] = num * y
            return carry

        lax.fori_loop(0, n_groups, group, 0)
        pltpu.sync_copy(out_v, out_hbm.at[pl.ds(base, b_per_w)])

    return sc_kernel


TP = 4096


def _tc_repack_body(t_ref, o_ref):
    blk = t_ref[...]
    o_ref[...] = blk.reshape(DIM, TP // 4, 4).transpose(1, 2, 0).reshape(
        TP // 4, 4 * DIM)


def _tc_repack(tableT):
    V = tableT.shape[1]
    grid = (V + TP - 1) // TP
    return pl.pallas_call(
        _tc_repack_body,
        grid=(grid,),
        in_specs=[pl.BlockSpec((DIM, TP), lambda j: (0, j))],
        out_specs=pl.BlockSpec((TP // 4, 4 * DIM), lambda j: (j, 0)),
        out_shape=jax.ShapeDtypeStruct((V // 4, 4 * DIM), jnp.float32),
    )(tableT)


def kernel(anchor_indices, source_indices, table, rel_vecs):
    B = source_indices.shape[0]
    V = table.shape[0]
    table_rm = _tc_repack(table.T).reshape(V, DIM)
    f = _build(B)
    return f(anchor_indices[0], anchor_indices[1], source_indices,
             table_rm, rel_vecs.reshape(-1))


# speedup vs baseline: 4.8309x; 4.8309x over previous
import functools

import jax
import jax.numpy as jnp
from jax import lax
from jax.experimental import pallas as pl
from jax.experimental.pallas import tpu as pltpu
from jax.experimental.pallas import tpu_sc as plsc

DIM = 32
L = 16
NW = 32
CHUNK = 128
PD = 128


def _build(B):
    b_per_w = B // NW
    n_chunks = b_per_w // CHUNK
    mesh = plsc.VectorSubcoreMesh(core_axis_name="c", subcore_axis_name="s")

    @functools.partial(
        pl.kernel,
        out_type=jax.ShapeDtypeStruct((B,), jnp.float32),
        mesh=mesh,
        scratch_types=[
            pltpu.VMEM((b_per_w,), jnp.int32),
            pltpu.VMEM((b_per_w,), jnp.int32),
            pltpu.VMEM((b_per_w,), jnp.int32),
            pltpu.VMEM((2, CHUNK, PD), jnp.float32),
            pltpu.VMEM((2, CHUNK, PD), jnp.float32),
            pltpu.VMEM((2, CHUNK, PD), jnp.float32),
            pltpu.VMEM((3 * DIM,), jnp.float32),
            pltpu.VMEM((b_per_w,), jnp.float32),
            pltpu.SemaphoreType.DMA,
        ],
        compiler_params=pltpu.CompilerParams(
            needs_layout_passes=False, use_tc_tiling_on_sc=False),
    )
    def sc_kernel(a0_hbm, a1_hbm, src_hbm, table_hbm, rel_hbm, out_hbm,
                  i0_v, i1_v, i2_v, r0_v, r1_v, r2_v, rv_v, out_v, sem):
        wid = lax.axis_index("s") * 2 + lax.axis_index("c")
        base = wid * b_per_w

        pltpu.sync_copy(a0_hbm.at[pl.ds(base, b_per_w)], i0_v)
        pltpu.sync_copy(a1_hbm.at[pl.ds(base, b_per_w)], i1_v)
        pltpu.sync_copy(src_hbm.at[pl.ds(base, b_per_w)], i2_v)
        pltpu.sync_copy(rel_hbm, rv_v)

        rd_lo = (rv_v[pl.ds(0, L)] * rv_v[pl.ds(DIM, L)]
                 * rv_v[pl.ds(2 * DIM, L)])
        rd_hi = (rv_v[pl.ds(L, L)] * rv_v[pl.ds(DIM + L, L)]
                 * rv_v[pl.ds(2 * DIM + L, L)])

        lane = lax.iota(jnp.int32, L)

        def fire(ch, buf):
            s = ch * CHUNK
            return [pltpu.async_copy(
                        table_hbm.at[iv.at[pl.ds(s, CHUNK)]],
                        rv.at[buf], sem)
                    for (iv, rv) in ((i0_v, r0_v), (i1_v, r1_v), (i2_v, r2_v))]

        def compute(ch, buf):
            def group(g, carry):
                base_e = g * L
                num = jnp.zeros((L,), jnp.float32)
                saa = jnp.zeros((L,), jnp.float32)
                see = jnp.zeros((L,), jnp.float32)
                for j in range(L):
                    e = base_e + j
                    a_lo = r0_v[buf, e, pl.ds(0, L)]
                    a_hi = r0_v[buf, e, pl.ds(L, L)]
                    b_lo = r1_v[buf, e, pl.ds(0, L)]
                    b_hi = r1_v[buf, e, pl.ds(L, L)]
                    t_lo = r2_v[buf, e, pl.ds(0, L)]
                    t_hi = r2_v[buf, e, pl.ds(L, L)]
                    u_lo = a_lo * b_lo
                    u_hi = a_hi * b_hi
                    w_lo = t_lo * rd_lo
                    w_hi = t_hi * rd_hi
                    num_s = jnp.sum(u_lo * w_lo + u_hi * w_hi)
                    saa_s = jnp.sum(w_lo * w_lo + w_hi * w_hi)
                    see_s = jnp.sum(u_lo * u_lo + u_hi * u_hi)
                    m = lane == j
                    num = jnp.where(m, num_s, num)
                    saa = jnp.where(m, saa_s, saa)
                    see = jnp.where(m, see_s, see)
                x = jnp.maximum(saa * see, jnp.float32(1e-24))
                i = plsc.bitcast(x, jnp.int32)
                y = plsc.bitcast(jnp.int32(0x5F3759DF) - (i >> 1), jnp.float32)
                for _ in range(3):
                    y = y * (jnp.float32(1.5) - jnp.float32(0.5) * x * y * y)
                out_v[pl.ds(ch * CHUNK + base_e, L)] = num * y
                return carry

            lax.fori_loop(0, CHUNK // L, group, 0)

        inflight = fire(0, 0)
        for ch in range(n_chunks):
            nxt = []
            if ch + 1 < n_chunks:
                nxt = fire(ch + 1, (ch + 1) % 2)
            for c in inflight:
                c.wait()
            compute(ch, ch % 2)
            inflight = nxt

        pltpu.sync_copy(out_v, out_hbm.at[pl.ds(base, b_per_w)])

    return sc_kernel


def kernel(anchor_indices, source_indices, table, rel_vecs):
    B = source_indices.shape[0]
    table_pad = jnp.pad(table, ((0, 0), (0, PD - DIM)))
    f = _build(B)
    return f(anchor_indices[0], anchor_indices[1], source_indices,
             table_pad, rel_vecs.reshape(-1))
